# baseline (speedup 1.0000x reference)
import functools

import jax
import jax.numpy as jnp
from jax import lax
from jax.experimental import pallas as pl
from jax.experimental.pallas import tpu as pltpu
from jax.experimental.pallas import tpu_sc as plsc

H = 256
NH = 10000
NF = 50000
E = 100000
ITERS = 4

NCORES = 2
NSUB = 16
LANES = 16
VPR = H // LANES
NW = NCORES * NSUB

WIN = 128
R_OUT = 512


def _nwin(n_dst):
    nblk = -(-n_dst // R_OUT)
    raw = -(-(E + nblk * WIN) // WIN)
    return -(-raw // NW) * NW


NWIN_HF = _nwin(NF)
NWIN_FH = _nwin(NH)


def _transform_body(h_ref, wa_ref, wb_ref, bb_ref, src_ref, dst_ref):
    h = h_ref[...]
    dn = (((1,), (1,)), ((), ()))
    src_ref[...] = lax.dot_general(h, wa_ref[...], dn,
                                   preferred_element_type=jnp.float32)
    yd = lax.dot_general(h, wb_ref[...], dn,
                         preferred_element_type=jnp.float32)
    dst_ref[...] = yd + bb_ref[...]


def _transform(h, wa, wb, bb, r):
    n = h.shape[0]
    grid = (n // r,)
    in_specs = [
        pl.BlockSpec((r, H), lambda i: (i, 0)),
        pl.BlockSpec((H, H), lambda i: (0, 0)),
        pl.BlockSpec((H, H), lambda i: (0, 0)),
        pl.BlockSpec((1, H), lambda i: (0, 0)),
    ]
    out_specs = [pl.BlockSpec((r, H), lambda i: (i, 0))] * 2
    out_shapes = [jax.ShapeDtypeStruct((n, H), jnp.float32),
                  jax.ShapeDtypeStruct((n + 8, H), jnp.float32)]
    return pl.pallas_call(
        _transform_body, grid=grid, in_specs=in_specs, out_specs=out_specs,
        out_shape=out_shapes)(h, wa, wb, bb)


def _gru_body(agg_ref, h_ref, wih_ref, whh_ref, bih_ref, bhh_ref, out_ref):
    x = agg_ref[...]
    h = h_ref[...]
    dn = (((1,), (1,)), ((), ()))
    gi = lax.dot_general(x, wih_ref[...], dn,
                         preferred_element_type=jnp.float32) + bih_ref[...]
    gh = lax.dot_general(h, whh_ref[...], dn,
                         preferred_element_type=jnp.float32) + bhh_ref[...]
    r = jax.nn.sigmoid(gi[:, 0:H] + gh[:, 0:H])
    z = jax.nn.sigmoid(gi[:, H:2 * H] + gh[:, H:2 * H])
    n = jnp.tanh(gi[:, 2 * H:] + r * gh[:, 2 * H:])
    out_ref[...] = (1.0 - z) * n + z * h


def _gru(agg, h, wih, whh, bih, bhh, r):
    n = h.shape[0]
    grid = (n // r,)
    in_specs = [
        pl.BlockSpec((r, H), lambda i: (i, 0)),
        pl.BlockSpec((r, H), lambda i: (i, 0)),
        pl.BlockSpec((3 * H, H), lambda i: (0, 0)),
        pl.BlockSpec((3 * H, H), lambda i: (0, 0)),
        pl.BlockSpec((1, 3 * H), lambda i: (0, 0)),
        pl.BlockSpec((1, 3 * H), lambda i: (0, 0)),
    ]
    return pl.pallas_call(
        _gru_body, grid=grid, in_specs=in_specs,
        out_specs=pl.BlockSpec((r, H), lambda i: (i, 0)),
        out_shape=jax.ShapeDtypeStruct((n, H), jnp.float32),
    )(agg, h, wih, whh, bih, bhh)


def _readout_body(h_ref, w1_ref, b1_ref, w2_ref, b2_ref, w3_ref, b3_ref,
                  out_ref):
    dn = (((1,), (1,)), ((), ()))
    h = h_ref[...]
    h1 = lax.dot_general(h, w1_ref[...], dn, preferred_element_type=jnp.float32)
    h1 = jnp.maximum(h1 + b1_ref[...], 0.0)
    h2 = lax.dot_general(h1, w2_ref[...], dn, preferred_element_type=jnp.float32)
    h2 = jnp.maximum(h2 + b2_ref[...], 0.0)
    h3 = lax.dot_general(h2, w3_ref[...], dn, preferred_element_type=jnp.float32)
    out_ref[...] = h3 + b3_ref[...]


def _readout(h, w1, b1, w2, b2, w3, b3, r):
    n = h.shape[0]
    nc = w3.shape[0]
    grid = (n // r,)
    in_specs = [
        pl.BlockSpec((r, H), lambda i: (i, 0)),
        pl.BlockSpec(w1.shape, lambda i: (0, 0)),
        pl.BlockSpec((1, w1.shape[0]), lambda i: (0, 0)),
        pl.BlockSpec(w2.shape, lambda i: (0, 0)),
        pl.BlockSpec((1, w2.shape[0]), lambda i: (0, 0)),
        pl.BlockSpec(w3.shape, lambda i: (0, 0)),
        pl.BlockSpec((1, nc), lambda i: (0, 0)),
    ]
    return pl.pallas_call(
        _readout_body, grid=grid, in_specs=in_specs,
        out_specs=pl.BlockSpec((r, nc), lambda i: (i, 0)),
        out_shape=jax.ShapeDtypeStruct((n, nc), jnp.float32),
    )(h, w1, b1.reshape(1, -1), w2, b2.reshape(1, -1), w3, b3.reshape(1, -1))


def _make_msg_kernel(nwin):
    nbw = nwin // NW

    mesh = plsc.VectorSubcoreMesh(
        core_axis_name="c", subcore_axis_name="s",
        num_cores=NCORES, num_subcores=NSUB)

    out_type = jax.ShapeDtypeStruct((nwin * WIN, H), jnp.float32)
    scratch = [
        pltpu.VMEM((nbw, WIN), jnp.int32),
        pltpu.VMEM((nbw, WIN), jnp.int32),
        pltpu.VMEM((WIN, H), jnp.float32),
        pltpu.VMEM((WIN, H), jnp.float32),
        pltpu.SemaphoreType.DMA,
        pltpu.SemaphoreType.DMA,
    ]

    @functools.partial(pl.kernel, out_type=out_type, mesh=mesh,
                       scratch_types=scratch)
    def k(st, dt, isrc_h, idst_h, out, idxs_v, idxd_v, ra, rb, sem0, sem1):
        c_id = lax.axis_index("c")
        s_id = lax.axis_index("s")
        w_id = s_id * NCORES + c_id

        cpi = pltpu.async_copy(isrc_h.at[w_id], idxs_v, sem0)
        cpj = pltpu.async_copy(idst_h.at[w_id], idxd_v, sem1)
        cpi.wait()
        cpj.wait()

        @pl.loop(0, nbw)
        def _batch(t):
            cpa = pltpu.async_copy(st.at[idxs_v.at[t]], ra, sem0)
            cpb = pltpu.async_copy(dt.at[idxd_v.at[t]], rb, sem1)
            cpa.wait()
            cpb.wait()

            @pl.loop(0, WIN)
            def _row(i):
                for j in range(VPR):
                    sl = pl.ds(j * LANES, LANES)
                    ra[i, sl] = jnp.maximum(ra[i, sl] + rb[i, sl], 0.0)

            pltpu.sync_copy(ra, out.at[pl.ds((w_id * nbw + t) * WIN, WIN)])

    return k


_msg_kernel_cache = {}


def _msg_kernel(nwin, *args):
    if nwin not in _msg_kernel_cache:
        _msg_kernel_cache[nwin] = _make_msg_kernel(nwin)
    return _msg_kernel_cache[nwin](*args)


def _reduce_body(n_dst, wb_ref, msg_ref, dst_ref, out_ref):
    i = pl.program_id(0)
    blk = wb_ref[i]
    dst = dst_ref[0, 0, :]
    loc = dst - blk * R_OUT
    oh = (lax.broadcasted_iota(jnp.int32, (R_OUT, WIN), 0)
          == loc[None, :]).astype(jnp.float32)
    msg = jnp.nan_to_num(msg_ref[...])
    contrib = lax.dot_general(oh, msg, (((1,), (0,)), ((), ())),
                              preferred_element_type=jnp.float32)
    prev = wb_ref[jnp.maximum(i - 1, 0)]
    is_first = jnp.logical_or(i == 0, blk != prev)

    @pl.when(is_first)
    def _init():
        out_ref[...] = contrib

    @pl.when(jnp.logical_not(is_first))
    def _acc():
        out_ref[...] += contrib


def _segment_reduce(msgs, dst3, wblk, n_dst):
    nwin = wblk.shape[0]
    nblk = -(-n_dst // R_OUT)
    grid_spec = pltpu.PrefetchScalarGridSpec(
        num_scalar_prefetch=1,
        grid=(nwin,),
        in_specs=[
            pl.BlockSpec((WIN, H), lambda i, wb: (i, 0)),
            pl.BlockSpec((1, 1, WIN), lambda i, wb: (i, 0, 0)),
        ],
        out_specs=pl.BlockSpec((R_OUT, H), lambda i, wb: (wb[i], 0)),
    )
    return pl.pallas_call(
        functools.partial(_reduce_body, n_dst), grid_spec=grid_spec,
        out_shape=jax.ShapeDtypeStruct((nblk * R_OUT, H), jnp.float32),
    )(wblk, msgs, dst3)


def _prep_edges(src_idx, dst_idx, n_dst, nwin):
    nblk = -(-n_dst // R_OUT)
    perm = jnp.argsort(dst_idx)
    ss = src_idx[perm]
    sd = dst_idx[perm]
    bnd = jnp.searchsorted(
        sd, jnp.arange(nblk + 1, dtype=jnp.int32) * R_OUT).astype(jnp.int32)
    cnt = bnd[1:] - bnd[:-1]
    pcnt = -(-cnt // WIN) * WIN
    poff = jnp.concatenate(
        [jnp.zeros((1,), jnp.int32), jnp.cumsum(pcnt).astype(jnp.int32)])
    blk = sd // R_OUT
    pos = poff[blk] + (jnp.arange(E, dtype=jnp.int32) - bnd[blk])
    epad = nwin * WIN
    es = jnp.zeros((epad,), jnp.int32).at[pos].set(ss)
    ed = jnp.full((epad,), n_dst, jnp.int32).at[pos].set(sd)
    wblk = jnp.clip(
        jnp.searchsorted(poff[1:],
                         jnp.arange(nwin, dtype=jnp.int32) * WIN,
                         side="right"),
        0, nblk - 1).astype(jnp.int32)
    nbw = nwin // NW
    es3 = es.reshape(NW, nbw, WIN)
    ed3 = ed.reshape(NW, nbw, WIN)
    dst3 = ed.reshape(nwin, 1, WIN)
    return es3, ed3, dst3, wblk


def kernel(host_features, flow_features, from_src, from_dst, to_src, to_dst,
           Wmf, bmf, Wmt, bmt,
           gh_Wih, gh_Whh, gh_bih, gh_bhh,
           gf_Wih, gf_Whh, gf_bih, gf_bhh,
           W1, b1, W2, b2, W3, b3):
    host_h = jnp.concatenate(
        [host_features,
         jnp.ones((NH, H - host_features.shape[1]), jnp.float32)], axis=1)
    flow_h = jnp.concatenate(
        [flow_features,
         jnp.zeros((NF, H - flow_features.shape[1]), jnp.float32)], axis=1)

    wmf_a = Wmf[:, :H]
    wmf_b = Wmf[:, H:]
    wmt_a = Wmt[:, :H]
    wmt_b = Wmt[:, H:]

    fs3, fd3, fdst3, fwblk = _prep_edges(from_src, from_dst, NF, NWIN_HF)
    ts3, td3, tdst3, twblk = _prep_edges(to_src, to_dst, NH, NWIN_FH)

    bmf2 = bmf.reshape(1, H)
    bmt2 = bmt.reshape(1, H)

    for _ in range(ITERS):
        host_src_t, host_dst_t = _transform(host_h, wmf_a, wmt_b, bmt2, 400)
        flow_src_t, flow_dst_t = _transform(flow_h, wmt_a, wmf_b, bmf2, 1000)

        msgs_f = _msg_kernel(NWIN_HF, host_src_t, flow_dst_t, fs3, fd3)
        msgs_h = _msg_kernel(NWIN_FH, flow_src_t, host_dst_t, ts3, td3)

        agg_flow = _segment_reduce(msgs_f, fdst3, fwblk, NF)
        agg_host = _segment_reduce(msgs_h, tdst3, twblk, NH)

        host_h = _gru(agg_host, host_h, gh_Wih, gh_Whh,
                      gh_bih.reshape(1, -1), gh_bhh.reshape(1, -1), 400)
        flow_h = _gru(agg_flow, flow_h, gf_Wih, gf_Whh,
                      gf_bih.reshape(1, -1), gf_bhh.reshape(1, -1), 1000)

    return _readout(flow_h, W1, b1, W2, b2, W3, b3, 1000)
